# baseline (speedup 1.0000x reference)
import functools

import jax
import jax.numpy as jnp
from jax import lax
from jax.experimental import pallas as pl
from jax.experimental.pallas import tpu as pltpu
from jax.experimental.pallas import tpu_sc as plsc

_VOCAB = 100000
_D = 768
_B = 4
_S = 4096
_N = _B * _S
_NW = 32
_SEQ_W = _S // _NW
_C = 32
_K = _SEQ_W // _C
_NT = _K * _B
_LANES = 16
_VECS = _D // _LANES


def _build_sc_kernel():
  mesh = plsc.VectorSubcoreMesh(core_axis_name="c", subcore_axis_name="s")

  @functools.partial(
      pl.kernel,
      mesh=mesh,
      out_type=jax.ShapeDtypeStruct((_N, _D), jnp.float32),
      scratch_types=[
          pltpu.VMEM((_B, _SEQ_W), jnp.int32),
          pltpu.VMEM((_C, _D), jnp.float32),
          pltpu.VMEM((_C, _D), jnp.float32),
          pltpu.VMEM((_C, _D), jnp.float32),
          pltpu.VMEM((_C, _D), jnp.float32),
          pltpu.VMEM((_C, _D), jnp.float32),
          pltpu.SemaphoreType.DMA,
          pltpu.SemaphoreType.DMA,
          pltpu.SemaphoreType.DMA,
          pltpu.SemaphoreType.DMA,
          pltpu.SemaphoreType.DMA,
          pltpu.SemaphoreType.DMA,
          pltpu.SemaphoreType.DMA,
      ],
  )
  def embed(x_hbm, tok_hbm, pos_hbm, out_hbm, idx_v,
            t0, t1, pb0, pb1, pb2, g0, g1, w0, w1, p0, p1, p2):
    tok = [t0, t1]
    pos_buf = [pb0, pb1, pb2]
    gsem = [g0, g1]
    wsem = [w0, w1]
    psem = [p0, p1, p2]

    cid = lax.axis_index("c")
    sid = lax.axis_index("s")
    wid = cid * 16 + sid
    seq_base = wid * _SEQ_W

    def start_pos(k):
      return pltpu.async_copy(
          pos_hbm.at[pl.ds(seq_base + k * _C, _C)], pos_buf[k % 3],
          psem[k % 3])

    def start_gather(t):
      k, bb = t // _B, t % _B
      return pltpu.async_copy(
          tok_hbm.at[idx_v.at[bb, pl.ds(k * _C, _C)]], tok[t % 2],
          gsem[t % 2])

    def start_write(t):
      k, bb = t // _B, t % _B
      return pltpu.async_copy(
          tok[t % 2], out_hbm.at[pl.ds(bb * _S + seq_base + k * _C, _C)],
          wsem[t % 2])

    def compute(t):
      k = t // _B

      @plsc.parallel_loop(0, _C)
      def row_body(r):
        for j in range(_VECS):
          sl = pl.ds(j * _LANES, _LANES)
          plsc.addupdate(tok[t % 2].at[r, sl], pos_buf[k % 3][r, sl])

    for bb in range(_B):
      pltpu.sync_copy(x_hbm.at[bb, pl.ds(seq_base, _SEQ_W)], idx_v.at[bb])
    ph = [start_pos(0), None, None]
    gh = {0: start_gather(0)}
    wh = {}

    for t in range(_NT):
      k = t // _B
      if t % _B == 0 and k + 1 < _K:
        ph[(k + 1) % 3] = start_pos(k + 1)
      if t + 1 < _NT:
        if t - 1 in wh:
          wh.pop(t - 1).wait()
        gh[t + 1] = start_gather(t + 1)
      gh.pop(t).wait()
      if t % _B == 0:
        ph[k % 3].wait()
      compute(t)
      wh[t] = start_write(t)

    for t in sorted(wh):
      wh.pop(t).wait()

  return embed


_EMBED = _build_sc_kernel()


def kernel(x, tok_table, pos_table):
  out = _EMBED(x.astype(jnp.int32), tok_table, pos_table)
  return out.reshape(_B, _S, _D)


# speedup vs baseline: 1.1752x; 1.1752x over previous
import functools

import jax
import jax.numpy as jnp
from jax import lax
from jax.experimental import pallas as pl
from jax.experimental.pallas import tpu as pltpu
from jax.experimental.pallas import tpu_sc as plsc

_VOCAB = 100000
_D = 768
_B = 4
_S = 4096
_N = _B * _S
_NW = 32
_SEQ_W = _S // _NW
_C = 16
_G = _SEQ_W // _C
_NT = _G * _B
_LANES = 16
_VECS = _D // _LANES


def _build_sc_kernel():
  mesh = plsc.VectorSubcoreMesh(core_axis_name="c", subcore_axis_name="s")

  @functools.partial(
      pl.kernel,
      mesh=mesh,
      out_type=jax.ShapeDtypeStruct((_N, _D), jnp.float32),
      scratch_types=[
          pltpu.VMEM((_B, _SEQ_W), jnp.int32),
          pltpu.VMEM((_C, _D), jnp.float32),
          pltpu.VMEM((_C, _D), jnp.float32),
          pltpu.VMEM((_C, _D), jnp.float32),
          pltpu.VMEM((_C, _D), jnp.float32),
          pltpu.VMEM((_C, _D), jnp.float32),
          pltpu.VMEM((_C, _D), jnp.float32),
          pltpu.SemaphoreType.DMA,
          pltpu.SemaphoreType.DMA,
          pltpu.SemaphoreType.DMA,
          pltpu.SemaphoreType.DMA,
          pltpu.SemaphoreType.DMA,
          pltpu.SemaphoreType.DMA,
          pltpu.SemaphoreType.DMA,
          pltpu.SemaphoreType.DMA,
          pltpu.SemaphoreType.DMA,
          pltpu.SemaphoreType.DMA,
      ],
  )
  def embed(x_hbm, tok_hbm, pos_hbm, out_hbm, idx_v,
            t0, t1, t2, t3, pb0, pb1,
            g0, g1, g2, g3, w0, w1, w2, w3, p0, p1):
    tok = [t0, t1, t2, t3]
    pos_buf = [pb0, pb1]
    gsem = [g0, g1, g2, g3]
    wsem = [w0, w1, w2, w3]
    psem = [p0, p1]

    cid = lax.axis_index("c")
    sid = lax.axis_index("s")
    wid = cid * 16 + sid
    seq_base = wid * _SEQ_W

    def pos_copy(g, p):
      return pltpu.make_async_copy(
          pos_hbm.at[pl.ds(seq_base + g * _C, _C)], pos_buf[p], psem[p])

    def gather_copy(g, i):
      return pltpu.make_async_copy(
          tok_hbm.at[idx_v.at[i % _B, pl.ds(g * _C, _C)]], tok[i % _B],
          gsem[i % _B])

    def write_copy(g, i):
      return pltpu.make_async_copy(
          tok[i % _B],
          out_hbm.at[pl.ds((i % _B) * _S + seq_base + g * _C, _C)],
          wsem[i % _B])

    def compute(i, p):
      @plsc.parallel_loop(0, _C)
      def row_body(r):
        for j in range(_VECS):
          sl = pl.ds(j * _LANES, _LANES)
          plsc.addupdate(tok[i % _B].at[r, sl], pos_buf[p][r, sl])

    for bb in range(_B):
      pltpu.sync_copy(x_hbm.at[bb, pl.ds(seq_base, _SEQ_W)], idx_v.at[bb])

    pos_copy(0, 0).start()
    gather_copy(0, 0).start()
    gather_copy(0, 1).start()

    def group_body(g, p):
      @pl.when(g < _G - 1)
      def _():
        pos_copy(g + 1, 1 - p).start()
      for i in range(_B):
        if i < 2:
          @pl.when(g > 0)
          def _():
            write_copy(g, i + 2).wait()
          gather_copy(g, i + 2).start()
        else:
          @pl.when(g < _G - 1)
          def _():
            write_copy(g, i - 2).wait()
            gather_copy(g + 1, i - 2).start()
        gather_copy(g, i).wait()
        if i == 0:
          pos_copy(g, p).wait()
        compute(i, p)
        write_copy(g, i).start()

    def pair_body(gg, carry):
      group_body(gg * 2, 0)
      group_body(gg * 2 + 1, 1)
      return carry

    lax.fori_loop(0, _G // 2, pair_body, 0)

    for i in range(_B):
      write_copy(_G - 1, i).wait()

  return embed


_EMBED = _build_sc_kernel()


def kernel(x, tok_table, pos_table):
  out = _EMBED(x.astype(jnp.int32), tok_table, pos_table)
  return out.reshape(_B, _S, _D)


# speedup vs baseline: 1.8096x; 1.5398x over previous
import functools

import jax
import jax.numpy as jnp
from jax import lax
from jax.experimental import pallas as pl
from jax.experimental.pallas import tpu as pltpu
from jax.experimental.pallas import tpu_sc as plsc

_VOCAB = 100000
_D = 768
_B = 4
_S = 4096
_N = _B * _S
_NW = 32
_SEQ_W = _S // _NW
_C = 32
_G = _SEQ_W // _C
_NT = _G * _B
_NBUF = 3
_LANES = 16
_VECS = _D // _LANES


def _build_sc_kernel():
  mesh = plsc.VectorSubcoreMesh(core_axis_name="c", subcore_axis_name="s")

  @functools.partial(
      pl.kernel,
      mesh=mesh,
      out_type=jax.ShapeDtypeStruct((_N, _D), jnp.float32),
      scratch_types=[
          pltpu.VMEM((_B, _SEQ_W), jnp.int32),
          pltpu.VMEM((_C, _D), jnp.float32),
          pltpu.VMEM((_C, _D), jnp.float32),
          pltpu.VMEM((_C, _D), jnp.float32),
          pltpu.VMEM((_C, _D), jnp.float32),
          pltpu.VMEM((_C, _D), jnp.float32),
          pltpu.SemaphoreType.DMA,
          pltpu.SemaphoreType.DMA,
          pltpu.SemaphoreType.DMA,
          pltpu.SemaphoreType.DMA,
          pltpu.SemaphoreType.DMA,
          pltpu.SemaphoreType.DMA,
          pltpu.SemaphoreType.DMA,
          pltpu.SemaphoreType.DMA,
      ],
  )
  def embed(x_hbm, tok_hbm, pos_hbm, out_hbm, idx_v,
            t0, t1, t2, pb0, pb1,
            g0, g1, g2, w0, w1, w2, p0, p1):
    tok = [t0, t1, t2]
    pos_buf = [pb0, pb1]
    gsem = [g0, g1, g2]
    wsem = [w0, w1, w2]
    psem = [p0, p1]

    cid = lax.axis_index("c")
    sid = lax.axis_index("s")
    wid = cid * 16 + sid
    seq_base = wid * _SEQ_W

    def pos_copy(k):
      return pltpu.make_async_copy(
          pos_hbm.at[pl.ds(seq_base + k * _C, _C)], pos_buf[k % 2],
          psem[k % 2])

    def gather_copy(t):
      k, bb, s = t // _B, t % _B, t % _NBUF
      return pltpu.make_async_copy(
          tok_hbm.at[idx_v.at[bb, pl.ds(k * _C, _C)]], tok[s], gsem[s])

    def write_copy(t):
      k, bb, s = t // _B, t % _B, t % _NBUF
      return pltpu.make_async_copy(
          tok[s], out_hbm.at[pl.ds(bb * _S + seq_base + k * _C, _C)],
          wsem[s])

    def compute(t):
      k, s = t // _B, t % _NBUF

      @plsc.parallel_loop(0, _C)
      def row_body(r):
        for j in range(_VECS):
          sl = pl.ds(j * _LANES, _LANES)
          plsc.addupdate(tok[s].at[r, sl], pos_buf[k % 2][r, sl])

    for bb in range(_B):
      pltpu.sync_copy(x_hbm.at[bb, pl.ds(seq_base, _SEQ_W)], idx_v.at[bb])
    pos_copy(0).start()
    gather_copy(0).start()
    gather_copy(1).start()

    for t in range(_NT):
      k = t // _B
      if t % _B == 0 and k + 1 < _G:
        pos_copy(k + 1).start()
      if t + 1 < _NT:
        if t - 2 >= 0:
          write_copy(t - 2).wait()
        gather_copy(t + 1).start()
      gather_copy(t).wait()
      if t % _B == 0:
        pos_copy(k).wait()
      compute(t)
      write_copy(t).start()

    write_copy(_NT - 2).wait()
    write_copy(_NT - 1).wait()

  return embed


_EMBED = _build_sc_kernel()


def kernel(x, tok_table, pos_table):
  out = _EMBED(x.astype(jnp.int32), tok_table, pos_table)
  return out.reshape(_B, _S, _D)
